# Initial kernel scaffold; baseline (speedup 1.0000x reference)
import jax
import jax.numpy as jnp
from jax.experimental import pallas as pl


def kernel(x, edge_index, edge_attr, coords, msg_W1, msg_b1, msg_W2, msg_b2, upd_W1, upd_b1, upd_W2, upd_b2, ln_g, ln_b):
    raise NotImplementedError("write your pallas kernel here")


# speedup vs baseline: 4.3017x; 4.3017x over previous
import functools

import jax
import jax.numpy as jnp
from jax import lax
from jax.experimental import pallas as pl
from jax.experimental.pallas import tpu as pltpu
from jax.experimental.pallas import tpu_sc as plsc

N = 10000
E = 320000
H = 128
EF = 16

NC = 2
NS = 16
NW = NC * NS
EPW = E // NW
CH = 80
NCHUNK = EPW // CH
RCP = 80
NRC = N // RCP
F32 = jnp.float32


def _prep_body(x_ref, c_ref, ws_ref, wd_ref, wc_ref, b1_ref, a_ref, b_ref):
    x = x_ref[...]
    cw = jnp.dot(c_ref[...], wc_ref[...], preferred_element_type=F32)
    a_ref[...] = jnp.dot(x, ws_ref[...], preferred_element_type=F32) - cw
    b_ref[...] = jnp.dot(x, wd_ref[...], preferred_element_type=F32) + cw + b1_ref[...]


def _prep(x, coords_p, ws, wd, wc_p, b1):
    blk = 1000
    grid = N // blk
    return pl.pallas_call(
        _prep_body,
        grid=(grid,),
        in_specs=[
            pl.BlockSpec((blk, H), lambda i: (i, 0)),
            pl.BlockSpec((blk, 8), lambda i: (i, 0)),
            pl.BlockSpec((H, H), lambda i: (0, 0)),
            pl.BlockSpec((H, H), lambda i: (0, 0)),
            pl.BlockSpec((8, H), lambda i: (0, 0)),
            pl.BlockSpec((1, H), lambda i: (0, 0)),
        ],
        out_specs=[
            pl.BlockSpec((blk, H), lambda i: (i, 0)),
            pl.BlockSpec((blk, H), lambda i: (i, 0)),
        ],
        out_shape=[
            jax.ShapeDtypeStruct((N, H), F32),
            jax.ShapeDtypeStruct((N, H), F32),
        ],
    )(x, coords_p, ws, wd, wc_p, b1)


def _gather_body(a_hbm, b_hbm, src_hbm, dst_hbm, g_hbm,
                 idxa, idxb, ra, rb, gv, sema, semb):
    wid = lax.axis_index("s") * NC + lax.axis_index("c")
    base = wid * EPW

    def chunk(t, carry):
        off = pl.multiple_of(base + t * CH, 8)
        pltpu.sync_copy(src_hbm.at[pl.ds(off, CH)], idxa)
        pltpu.sync_copy(dst_hbm.at[pl.ds(off, CH)], idxb)
        cpa = pltpu.async_copy(a_hbm.at[idxa], ra, sema)
        cpb = pltpu.async_copy(b_hbm.at[idxb], rb, semb)
        cpa.wait()
        cpb.wait()

        def row(e, c2):
            for k in range(8):
                sl = pl.ds(k * 16, 16)
                gv[e, sl] = ra[e, sl] + rb[e, sl]
            return c2

        lax.fori_loop(0, CH, row, 0)
        pltpu.sync_copy(gv, g_hbm.at[pl.ds(off, CH)])
        return carry

    lax.fori_loop(0, NCHUNK, chunk, 0)


_gather = functools.partial(
    pl.kernel,
    out_type=jax.ShapeDtypeStruct((E, H), F32),
    mesh=plsc.VectorSubcoreMesh(core_axis_name="c", subcore_axis_name="s",
                                num_cores=NC, num_subcores=NS),
    scratch_types=[
        pltpu.VMEM((CH,), jnp.int32),
        pltpu.VMEM((CH,), jnp.int32),
        pltpu.VMEM((CH, H), F32),
        pltpu.VMEM((CH, H), F32),
        pltpu.VMEM((CH, H), F32),
        pltpu.SemaphoreType.DMA,
        pltpu.SemaphoreType.DMA,
    ],
)(_gather_body)


def _edge_body(g_ref, ea_ref, we_ref, w2_ref, b2_ref, m_ref):
    h1 = jnp.maximum(
        g_ref[...] + jnp.dot(ea_ref[...], we_ref[...], preferred_element_type=F32),
        0.0)
    m_ref[...] = jnp.maximum(
        jnp.dot(h1, w2_ref[...], preferred_element_type=F32) + b2_ref[...],
        0.0)


def _edge(g, ea, we, w2, b2):
    blk = 3200
    grid = E // blk
    return pl.pallas_call(
        _edge_body,
        grid=(grid,),
        in_specs=[
            pl.BlockSpec((blk, H), lambda i: (i, 0)),
            pl.BlockSpec((blk, EF), lambda i: (i, 0)),
            pl.BlockSpec((EF, H), lambda i: (0, 0)),
            pl.BlockSpec((H, H), lambda i: (0, 0)),
            pl.BlockSpec((1, H), lambda i: (0, 0)),
        ],
        out_specs=pl.BlockSpec((blk, H), lambda i: (i, 0)),
        out_shape=jax.ShapeDtypeStruct((E, H), F32),
    )(g, ea, we, w2, b2)


def _scatter_body(msg_hbm, dst_hbm, out_hbm, idxv, mv, zv, accum, sem_unused):
    cid = lax.axis_index("c")
    sid = lax.axis_index("s")
    base = cid * (E // NC) + sid * EPW

    def zrow(r, c2):
        for k in range(8):
            zv[r, pl.ds(k * 16, 16)] = jnp.zeros((16,), F32)
        return c2

    lax.fori_loop(0, RCP, zrow, 0)

    nmine = (NRC - 1 - sid) // NS + 1

    def zcopy(j, c2):
        r0 = pl.multiple_of((sid + j * NS) * RCP, 8)
        pltpu.sync_copy(zv, accum.at[pl.ds(r0, RCP)])
        return c2

    lax.fori_loop(0, nmine, zcopy, 0)
    plsc.subcore_barrier()

    def chunk(t, c2):
        off = pl.multiple_of(base + t * CH, 8)
        pltpu.sync_copy(dst_hbm.at[pl.ds(off, CH)], idxv)
        pltpu.sync_copy(msg_hbm.at[pl.ds(off, CH)], mv)
        pltpu.sync_copy(mv, accum.at[idxv], add=True)
        return c2

    lax.fori_loop(0, NCHUNK, chunk, 0)
    plsc.subcore_barrier()

    def ocopy(j, c2):
        r0 = pl.multiple_of((sid + j * NS) * RCP, 8)
        pltpu.sync_copy(accum.at[pl.ds(r0, RCP)],
                        out_hbm.at[pl.ds(pl.multiple_of(cid * N + r0, 8), RCP)])
        return c2

    lax.fori_loop(0, nmine, ocopy, 0)


_scatter = functools.partial(
    pl.kernel,
    out_type=jax.ShapeDtypeStruct((2 * N, H), F32),
    mesh=plsc.VectorSubcoreMesh(core_axis_name="c", subcore_axis_name="s",
                                num_cores=NC, num_subcores=NS),
    scratch_types=[
        pltpu.VMEM((CH,), jnp.int32),
        pltpu.VMEM((CH, H), F32),
        pltpu.VMEM((RCP, H), F32),
        pltpu.VMEM_SHARED((N, H), F32),
        pltpu.SemaphoreType.DMA,
    ],
)(_scatter_body)


def _node_body(x_ref, p_ref, u1a_ref, u1b_ref, b1_ref, u2_ref, b2_ref,
               g_ref, bb_ref, o_ref):
    x = x_ref[...]
    aggr = p_ref[0, :, :] + p_ref[1, :, :]
    h = jnp.maximum(
        jnp.dot(x, u1a_ref[...], preferred_element_type=F32)
        + jnp.dot(aggr, u1b_ref[...], preferred_element_type=F32)
        + b1_ref[...], 0.0)
    o = jnp.maximum(jnp.dot(h, u2_ref[...], preferred_element_type=F32)
                    + b2_ref[...], 0.0)
    y = x + o
    mu = jnp.mean(y, axis=1, keepdims=True)
    var = jnp.mean((y - mu) * (y - mu), axis=1, keepdims=True)
    o_ref[...] = (y - mu) * lax.rsqrt(var + 1e-5) * g_ref[...] + bb_ref[...]


def _node(x, p, u1a, u1b, b1, u2, b2, g, b):
    blk = 1000
    grid = N // blk
    return pl.pallas_call(
        _node_body,
        grid=(grid,),
        in_specs=[
            pl.BlockSpec((blk, H), lambda i: (i, 0)),
            pl.BlockSpec((2, blk, H), lambda i: (0, i, 0)),
            pl.BlockSpec((H, H), lambda i: (0, 0)),
            pl.BlockSpec((H, H), lambda i: (0, 0)),
            pl.BlockSpec((1, H), lambda i: (0, 0)),
            pl.BlockSpec((H, H), lambda i: (0, 0)),
            pl.BlockSpec((1, H), lambda i: (0, 0)),
            pl.BlockSpec((1, H), lambda i: (0, 0)),
            pl.BlockSpec((1, H), lambda i: (0, 0)),
        ],
        out_specs=pl.BlockSpec((blk, H), lambda i: (i, 0)),
        out_shape=jax.ShapeDtypeStruct((N, H), F32),
    )(x, p, u1a, u1b, b1, u2, b2, g, b)


def kernel(x, edge_index, edge_attr, coords,
           msg_W1, msg_b1, msg_W2, msg_b2,
           upd_W1, upd_b1, upd_W2, upd_b2,
           ln_g, ln_b):
    src = edge_index[0]
    dst = edge_index[1]
    ws = msg_W1[:H]
    wd = msg_W1[H:2 * H]
    we = msg_W1[2 * H:2 * H + EF]
    wc = msg_W1[2 * H + EF:]
    coords_p = jnp.pad(coords, ((0, 0), (0, 5)))
    wc_p = jnp.pad(wc, ((0, 5), (0, 0)))

    a, b = _prep(x, coords_p, ws, wd, wc_p, msg_b1.reshape(1, H))
    g = _gather(a, b, src, dst)
    msg = _edge(g, edge_attr, we, msg_W2, msg_b2.reshape(1, H))
    p = _scatter(msg, dst).reshape(2, N, H)
    return _node(x, p, upd_W1[:H], upd_W1[H:], upd_b1.reshape(1, H),
                 upd_W2, upd_b2.reshape(1, H),
                 ln_g.reshape(1, H), ln_b.reshape(1, H))


# speedup vs baseline: 4.5362x; 1.0545x over previous
import functools

import jax
import jax.numpy as jnp
from jax import lax
from jax.experimental import pallas as pl
from jax.experimental.pallas import tpu as pltpu
from jax.experimental.pallas import tpu_sc as plsc

N = 10000
E = 320000
H = 128
EF = 16

NC = 2
NS = 16
NW = NC * NS
EPW = E // NW
CH = 80
NCHUNK = EPW // CH
RCP = 80
NRC = N // RCP
F32 = jnp.float32


def _prep_body(x_ref, c_ref, ws_ref, wd_ref, wc_ref, b1_ref, a_ref, b_ref):
    x = x_ref[...]
    cw = jnp.dot(c_ref[...], wc_ref[...], preferred_element_type=F32)
    a_ref[...] = jnp.dot(x, ws_ref[...], preferred_element_type=F32) - cw
    b_ref[...] = jnp.dot(x, wd_ref[...], preferred_element_type=F32) + cw + b1_ref[...]


def _prep(x, coords_p, ws, wd, wc_p, b1):
    blk = 1000
    grid = N // blk
    return pl.pallas_call(
        _prep_body,
        grid=(grid,),
        in_specs=[
            pl.BlockSpec((blk, H), lambda i: (i, 0)),
            pl.BlockSpec((blk, 8), lambda i: (i, 0)),
            pl.BlockSpec((H, H), lambda i: (0, 0)),
            pl.BlockSpec((H, H), lambda i: (0, 0)),
            pl.BlockSpec((8, H), lambda i: (0, 0)),
            pl.BlockSpec((1, H), lambda i: (0, 0)),
        ],
        out_specs=[
            pl.BlockSpec((blk, H), lambda i: (i, 0)),
            pl.BlockSpec((blk, H), lambda i: (i, 0)),
        ],
        out_shape=[
            jax.ShapeDtypeStruct((N, H), F32),
            jax.ShapeDtypeStruct((N, H), F32),
        ],
    )(x, coords_p, ws, wd, wc_p, b1)


CHG = 40
NCG = EPW // CHG


def _gather_body(a_hbm, b_hbm, src_hbm, dst_hbm, g_hbm, *s):
    idxa = s[0:2]
    idxb = s[2:4]
    ra = s[4:6]
    rb = s[6:8]
    gv = s[8:10]
    sia = s[10:12]
    sib = s[12:14]
    sa = s[14:16]
    sb = s[16:18]
    sw = s[18:20]
    wid = lax.axis_index("s") * NC + lax.axis_index("c")
    base = wid * EPW

    def off_of(t):
        return pl.multiple_of(base + t * CHG, 8)

    for b in range(2):
        off = off_of(b)
        pltpu.make_async_copy(src_hbm.at[pl.ds(off, CHG)], idxa[b], sia[b]).start()
        pltpu.make_async_copy(dst_hbm.at[pl.ds(off, CHG)], idxb[b], sib[b]).start()

    def outer(g, carry):
        for b in range(2):
            t = 2 * g + b
            off = off_of(t)
            pltpu.make_async_copy(src_hbm.at[pl.ds(off, CHG)], idxa[b], sia[b]).wait()
            pltpu.make_async_copy(dst_hbm.at[pl.ds(off, CHG)], idxb[b], sib[b]).wait()
            pltpu.make_async_copy(a_hbm.at[idxa[b]], ra[b], sa[b]).start()
            pltpu.make_async_copy(b_hbm.at[idxb[b]], rb[b], sb[b]).start()
        for b in range(2):
            t = 2 * g + b
            off = off_of(t)
            pltpu.make_async_copy(a_hbm.at[idxa[b]], ra[b], sa[b]).wait()
            pltpu.make_async_copy(b_hbm.at[idxb[b]], rb[b], sb[b]).wait()
            ra_b, rb_b, gv_b = ra[b], rb[b], gv[b]

            @pl.when(g > 0)
            def _():
                poff = off_of(t - 2)
                pltpu.make_async_copy(gv_b, g_hbm.at[pl.ds(poff, CHG)], sw[b]).wait()

            def row(e, c2):
                for k in range(8):
                    sl = pl.ds(k * 16, 16)
                    gv_b[e, sl] = ra_b[e, sl] + rb_b[e, sl]
                return c2

            lax.fori_loop(0, CHG, row, 0, unroll=2)
            pltpu.make_async_copy(gv_b, g_hbm.at[pl.ds(off, CHG)], sw[b]).start()

            @pl.when(t + 2 < NCG)
            def _():
                noff = off_of(t + 2)
                pltpu.make_async_copy(src_hbm.at[pl.ds(noff, CHG)], idxa[b], sia[b]).start()
                pltpu.make_async_copy(dst_hbm.at[pl.ds(noff, CHG)], idxb[b], sib[b]).start()
        return carry

    lax.fori_loop(0, NCG // 2, outer, 0)
    for b in range(2):
        off = off_of(NCG - 2 + b)
        pltpu.make_async_copy(gv[b], g_hbm.at[pl.ds(off, CHG)], sw[b]).wait()


_gather = functools.partial(
    pl.kernel,
    out_type=jax.ShapeDtypeStruct((E, H), F32),
    mesh=plsc.VectorSubcoreMesh(core_axis_name="c", subcore_axis_name="s",
                                num_cores=NC, num_subcores=NS),
    scratch_types=(
        [pltpu.VMEM((CHG,), jnp.int32) for _ in range(4)]
        + [pltpu.VMEM((CHG, H), F32) for _ in range(6)]
        + [pltpu.SemaphoreType.DMA for _ in range(10)]
    ),
)(_gather_body)


def _edge_body(g_ref, ea_ref, we_ref, w2_ref, b2_ref, m_ref):
    h1 = jnp.maximum(
        g_ref[...] + jnp.dot(ea_ref[...], we_ref[...], preferred_element_type=F32),
        0.0)
    m_ref[...] = jnp.maximum(
        jnp.dot(h1, w2_ref[...], preferred_element_type=F32) + b2_ref[...],
        0.0)


def _edge(g, ea, we, w2, b2):
    blk = 3200
    grid = E // blk
    return pl.pallas_call(
        _edge_body,
        grid=(grid,),
        in_specs=[
            pl.BlockSpec((blk, H), lambda i: (i, 0)),
            pl.BlockSpec((blk, EF), lambda i: (i, 0)),
            pl.BlockSpec((EF, H), lambda i: (0, 0)),
            pl.BlockSpec((H, H), lambda i: (0, 0)),
            pl.BlockSpec((1, H), lambda i: (0, 0)),
        ],
        out_specs=pl.BlockSpec((blk, H), lambda i: (i, 0)),
        out_shape=jax.ShapeDtypeStruct((E, H), F32),
    )(g, ea, we, w2, b2)


CHS = 40
NCS = EPW // CHS
NSL = 5


def _scatter_body(msg_hbm, dst_hbm, out_hbm, *s):
    idxs = s[0:NSL]
    mv = s[NSL:2 * NSL]
    zv = s[2 * NSL]
    accum = s[2 * NSL + 1]
    sli = s[2 * NSL + 2:2 * NSL + 2 + NSL]
    slm = s[2 * NSL + 2 + NSL:2 * NSL + 2 + 2 * NSL]
    sad = s[2 * NSL + 2 + 2 * NSL:2 * NSL + 2 + 3 * NSL]
    cid = lax.axis_index("c")
    sid = lax.axis_index("s")
    base = cid * (E // NC) + sid * EPW

    def off_of(t):
        return pl.multiple_of(base + t * CHS, 8)

    def zrow(r, c2):
        for k in range(8):
            zv[r, pl.ds(k * 16, 16)] = jnp.zeros((16,), F32)
        return c2

    lax.fori_loop(0, RCP, zrow, 0)

    nmine = (NRC - 1 - sid) // NS + 1

    def zcopy(j, c2):
        r0 = pl.multiple_of((sid + j * NS) * RCP, 8)
        pltpu.sync_copy(zv, accum.at[pl.ds(r0, RCP)])
        return c2

    lax.fori_loop(0, nmine, zcopy, 0)
    plsc.subcore_barrier()

    for b in range(NSL):
        off = off_of(b)
        pltpu.make_async_copy(dst_hbm.at[pl.ds(off, CHS)], idxs[b], sli[b]).start()
        pltpu.make_async_copy(msg_hbm.at[pl.ds(off, CHS)], mv[b], slm[b]).start()

    def outer(g, carry):
        for b in range(NSL):
            t = NSL * g + b
            off = off_of(t)
            pltpu.make_async_copy(dst_hbm.at[pl.ds(off, CHS)], idxs[b], sli[b]).wait()
            pltpu.make_async_copy(msg_hbm.at[pl.ds(off, CHS)], mv[b], slm[b]).wait()
            pltpu.async_copy(mv[b], accum.at[idxs[b]], sad[b], add=True)
        for b in range(NSL):
            t = NSL * g + b

            @pl.when(t + NSL < NCS)
            def _():
                noff = off_of(t + NSL)
                pltpu.make_async_copy(mv[b], accum.at[idxs[b]], sad[b]).wait()
                pltpu.make_async_copy(dst_hbm.at[pl.ds(noff, CHS)], idxs[b], sli[b]).start()
                pltpu.make_async_copy(msg_hbm.at[pl.ds(noff, CHS)], mv[b], slm[b]).start()
        return carry

    lax.fori_loop(0, NCS // NSL, outer, 0)
    for b in range(NSL):
        pltpu.make_async_copy(mv[b], accum.at[idxs[b]], sad[b]).wait()
    plsc.subcore_barrier()

    def ocopy(j, c2):
        r0 = pl.multiple_of((sid + j * NS) * RCP, 8)
        pltpu.sync_copy(accum.at[pl.ds(r0, RCP)],
                        out_hbm.at[pl.ds(pl.multiple_of(cid * N + r0, 8), RCP)])
        return c2

    lax.fori_loop(0, nmine, ocopy, 0)


_scatter = functools.partial(
    pl.kernel,
    out_type=jax.ShapeDtypeStruct((2 * N, H), F32),
    mesh=plsc.VectorSubcoreMesh(core_axis_name="c", subcore_axis_name="s",
                                num_cores=NC, num_subcores=NS),
    scratch_types=(
        [pltpu.VMEM((CHS,), jnp.int32) for _ in range(NSL)]
        + [pltpu.VMEM((CHS, H), F32) for _ in range(NSL)]
        + [pltpu.VMEM((RCP, H), F32), pltpu.VMEM_SHARED((N, H), F32)]
        + [pltpu.SemaphoreType.DMA for _ in range(3 * NSL)]
    ),
)(_scatter_body)


def _node_body(x_ref, p_ref, u1a_ref, u1b_ref, b1_ref, u2_ref, b2_ref,
               g_ref, bb_ref, o_ref):
    x = x_ref[...]
    aggr = p_ref[0, :, :] + p_ref[1, :, :]
    h = jnp.maximum(
        jnp.dot(x, u1a_ref[...], preferred_element_type=F32)
        + jnp.dot(aggr, u1b_ref[...], preferred_element_type=F32)
        + b1_ref[...], 0.0)
    o = jnp.maximum(jnp.dot(h, u2_ref[...], preferred_element_type=F32)
                    + b2_ref[...], 0.0)
    y = x + o
    mu = jnp.mean(y, axis=1, keepdims=True)
    var = jnp.mean((y - mu) * (y - mu), axis=1, keepdims=True)
    o_ref[...] = (y - mu) * lax.rsqrt(var + 1e-5) * g_ref[...] + bb_ref[...]


def _node(x, p, u1a, u1b, b1, u2, b2, g, b):
    blk = 1000
    grid = N // blk
    return pl.pallas_call(
        _node_body,
        grid=(grid,),
        in_specs=[
            pl.BlockSpec((blk, H), lambda i: (i, 0)),
            pl.BlockSpec((2, blk, H), lambda i: (0, i, 0)),
            pl.BlockSpec((H, H), lambda i: (0, 0)),
            pl.BlockSpec((H, H), lambda i: (0, 0)),
            pl.BlockSpec((1, H), lambda i: (0, 0)),
            pl.BlockSpec((H, H), lambda i: (0, 0)),
            pl.BlockSpec((1, H), lambda i: (0, 0)),
            pl.BlockSpec((1, H), lambda i: (0, 0)),
            pl.BlockSpec((1, H), lambda i: (0, 0)),
        ],
        out_specs=pl.BlockSpec((blk, H), lambda i: (i, 0)),
        out_shape=jax.ShapeDtypeStruct((N, H), F32),
    )(x, p, u1a, u1b, b1, u2, b2, g, b)


def kernel(x, edge_index, edge_attr, coords,
           msg_W1, msg_b1, msg_W2, msg_b2,
           upd_W1, upd_b1, upd_W2, upd_b2,
           ln_g, ln_b):
    src = edge_index[0]
    dst = edge_index[1]
    ws = msg_W1[:H]
    wd = msg_W1[H:2 * H]
    we = msg_W1[2 * H:2 * H + EF]
    wc = msg_W1[2 * H + EF:]
    coords_p = jnp.pad(coords, ((0, 0), (0, 5)))
    wc_p = jnp.pad(wc, ((0, 5), (0, 0)))

    a, b = _prep(x, coords_p, ws, wd, wc_p, msg_b1.reshape(1, H))
    g = _gather(a, b, src, dst)
    msg = _edge(g, edge_attr, we, msg_W2, msg_b2.reshape(1, H))
    p = _scatter(msg, dst).reshape(2, N, H)
    return _node(x, p, upd_W1[:H], upd_W1[H:], upd_b1.reshape(1, H),
                 upd_W2, upd_b2.reshape(1, H),
                 ln_g.reshape(1, H), ln_b.reshape(1, H))
